# Initial kernel scaffold; baseline (speedup 1.0000x reference)
import functools

import jax
import jax.numpy as jnp
from jax import lax
from jax.experimental import pallas as pl
from jax.experimental.pallas import tpu as pltpu
from jax.experimental.pallas import tpu_sc as plsc

_B = 4096
_L = 50
_E = 32
_N = _B * _L
_LBL = 1000
_LBL_PAD = 1024

_NC = 2
_NS = 16
_NW = _NC * _NS
_PER_W = _N // _NW
_CH = 800
_NCH = _PER_W // _CH


def _sc_gather_body(idx_hbm, table_hbm, out_hbm, idx_v, rows_v, gs0, gs1, ss0, ss1):
    gs = (gs0, gs1)
    ss = (ss0, ss1)
    wid = lax.axis_index("s") * _NC + lax.axis_index("c")
    base_chunk = wid * _NCH

    pltpu.sync_copy(idx_hbm.at[pl.ds(base_chunk, _NCH)], idx_v)

    def fire_gather(j):
        pltpu.async_copy(table_hbm.at[idx_v.at[j]], rows_v.at[j % 2], gs[j % 2])

    def fire_store(j):
        row0 = (base_chunk + j) * _CH
        pltpu.async_copy(rows_v.at[j % 2], out_hbm.at[pl.ds(row0, _CH)], ss[j % 2])

    fire_gather(0)
    fire_gather(1)
    for j in range(_NCH):
        k = j % 2
        pltpu.make_async_copy(table_hbm.at[idx_v.at[j]], rows_v.at[k], gs[k]).wait()
        fire_store(j)
        if j + 2 < _NCH:
            row0 = (base_chunk + j) * _CH
            pltpu.make_async_copy(rows_v.at[k], out_hbm.at[pl.ds(row0, _CH)], ss[k]).wait()
            fire_gather(j + 2)
    for j in (_NCH - 2, _NCH - 1):
        row0 = (base_chunk + j) * _CH
        pltpu.make_async_copy(rows_v.at[j % 2], out_hbm.at[pl.ds(row0, _CH)], ss[j % 2]).wait()


def _sc_gather(idx2d, table):
    mesh = plsc.VectorSubcoreMesh(core_axis_name="c", subcore_axis_name="s")
    fn = pl.kernel(
        _sc_gather_body,
        out_type=jax.ShapeDtypeStruct((_N, _E), jnp.float32),
        mesh=mesh,
        scratch_types=[
            pltpu.VMEM((_NCH, _CH), jnp.int32),
            pltpu.VMEM((2, _CH, _E), jnp.float32),
            pltpu.SemaphoreType.DMA,
            pltpu.SemaphoreType.DMA,
            pltpu.SemaphoreType.DMA,
            pltpu.SemaphoreType.DMA,
        ],
    )
    return fn(idx2d, table)


def _head_body(flat_ref, wt_ref, bias_ref, out_ref):
    a = flat_ref[...].astype(jnp.bfloat16)
    logits = jnp.dot(a, wt_ref[...], preferred_element_type=jnp.float32)
    logits = logits + bias_ref[...]
    m = jnp.max(logits, axis=-1, keepdims=True)
    e = jnp.exp(logits - m)
    s = jnp.sum(e, axis=-1, keepdims=True)
    out_ref[...] = logits - m - jnp.log(s)


def _tc_head(flat, wt, bias):
    bs = 512
    grid = (_B // bs,)
    return pl.pallas_call(
        _head_body,
        grid=grid,
        in_specs=[
            pl.BlockSpec((bs, _L * _E), lambda i: (i, 0)),
            pl.BlockSpec((_L * _E, _LBL_PAD), lambda i: (0, 0)),
            pl.BlockSpec((1, _LBL_PAD), lambda i: (0, 0)),
        ],
        out_specs=pl.BlockSpec((bs, _LBL_PAD), lambda i: (i, 0)),
        out_shape=jax.ShapeDtypeStruct((_B, _LBL_PAD), jnp.float32),
    )(flat, wt, bias)


def kernel(x, table, W, b):
    idx2d = x.reshape(_N // _CH, _CH).astype(jnp.int32)
    emb = _sc_gather(idx2d, table)
    flat = emb.reshape(_B, _L * _E)
    wt = jnp.concatenate(
        [W.T, jnp.zeros((_L * _E, _LBL_PAD - _LBL), W.dtype)], axis=1
    ).astype(jnp.bfloat16)
    bias = jnp.concatenate([b, jnp.full((_LBL_PAD - _LBL,), -1e30, b.dtype)])
    out = _tc_head(flat, wt, bias.reshape(1, _LBL_PAD))
    return out[:, :_LBL]


# speedup vs baseline: 8.6221x; 8.6221x over previous
import functools

import jax
import jax.numpy as jnp
from jax import lax
from jax.experimental import pallas as pl
from jax.experimental.pallas import tpu as pltpu
from jax.experimental.pallas import tpu_sc as plsc

_B = 4096
_L = 50
_E = 32
_N = _B * _L
_LBL = 1000
_LBL_PAD = 1024

_NC = 2
_NS = 16
_NW = _NC * _NS
_PER_W = _N // _NW
_CH = 128
_NCH = _PER_W // _CH
_PAIRS = _NCH // 2


def _sc_gather_body(idx_hbm, table_hbm, out_hbm, idx_v, rows_v, gs0, gs1, ss0, ss1):
    wid = lax.axis_index("s") * _NC + lax.axis_index("c")
    base = wid * _PER_W

    pltpu.sync_copy(idx_hbm.at[pl.ds(base, _PER_W)], idx_v)

    def idx_slice(j):
        return idx_v.at[pl.ds(pl.multiple_of(j * _CH, _CH), _CH)]

    def fire_gather(j, buf, sem):
        pltpu.async_copy(table_hbm.at[idx_slice(j)], rows_v.at[buf], sem)

    def wait_gather(j, buf, sem):
        pltpu.make_async_copy(table_hbm.at[idx_slice(j)], rows_v.at[buf], sem).wait()

    def store_descr(j, buf, sem):
        row0 = pl.multiple_of(base + j * _CH, _CH)
        return pltpu.make_async_copy(
            rows_v.at[buf], out_hbm.at[pl.ds(row0, _CH)], sem)

    fire_gather(0, 0, gs0)
    fire_gather(1, 1, gs1)

    def step(jj, carry):
        j0 = jj * 2
        j1 = j0 + 1
        wait_gather(j0, 0, gs0)
        store_descr(j0, 0, ss0).start()
        wait_gather(j1, 1, gs1)
        store_descr(j1, 1, ss1).start()

        @pl.when(jj < _PAIRS - 1)
        def _():
            store_descr(j0, 0, ss0).wait()
            fire_gather(j0 + 2, 0, gs0)
            store_descr(j1, 1, ss1).wait()
            fire_gather(j1 + 2, 1, gs1)

        return carry

    lax.fori_loop(0, _PAIRS, step, 0)
    store_descr(_NCH - 2, 0, ss0).wait()
    store_descr(_NCH - 1, 1, ss1).wait()


def _sc_gather(idx2d, table):
    mesh = plsc.VectorSubcoreMesh(core_axis_name="c", subcore_axis_name="s")
    fn = pl.kernel(
        _sc_gather_body,
        out_type=jax.ShapeDtypeStruct((_N, _E), jnp.float32),
        mesh=mesh,
        scratch_types=[
            pltpu.VMEM((_PER_W,), jnp.int32),
            pltpu.VMEM((2, _CH, _E), jnp.float32),
            pltpu.SemaphoreType.DMA,
            pltpu.SemaphoreType.DMA,
            pltpu.SemaphoreType.DMA,
            pltpu.SemaphoreType.DMA,
        ],
        compiler_params=pltpu.CompilerParams(use_tc_tiling_on_sc=False),
    )
    return fn(idx2d, table)


def _head_body(flat_ref, wt_ref, bias_ref, out_ref):
    a = flat_ref[...].astype(jnp.bfloat16)
    logits = jnp.dot(a, wt_ref[...], preferred_element_type=jnp.float32)
    logits = logits + bias_ref[...]
    m = jnp.max(logits, axis=-1, keepdims=True)
    e = jnp.exp(logits - m)
    s = jnp.sum(e, axis=-1, keepdims=True)
    out_ref[...] = logits - m - jnp.log(s)


def _tc_head(flat, wt, bias):
    bs = 512
    grid = (_B // bs,)
    return pl.pallas_call(
        _head_body,
        grid=grid,
        in_specs=[
            pl.BlockSpec((bs, _L * _E), lambda i: (i, 0)),
            pl.BlockSpec((_L * _E, _LBL_PAD), lambda i: (0, 0)),
            pl.BlockSpec((1, _LBL_PAD), lambda i: (0, 0)),
        ],
        out_specs=pl.BlockSpec((bs, _LBL_PAD), lambda i: (i, 0)),
        out_shape=jax.ShapeDtypeStruct((_B, _LBL_PAD), jnp.float32),
    )(flat, wt, bias)


def kernel(x, table, W, b):
    idx1d = x.reshape(_N).astype(jnp.int32)
    emb = _sc_gather(idx1d, table)
    flat = emb.reshape(_B, _L * _E)
    wt = jnp.concatenate(
        [W.T, jnp.zeros((_L * _E, _LBL_PAD - _LBL), W.dtype)], axis=1
    ).astype(jnp.bfloat16)
    bias = jnp.concatenate([b, jnp.full((_LBL_PAD - _LBL,), -1e30, b.dtype)])
    out = _tc_head(flat, wt, bias.reshape(1, _LBL_PAD))
    return out[:, :_LBL]
